# Initial kernel scaffold; baseline (speedup 1.0000x reference)
import jax
import jax.numpy as jnp
from jax.experimental import pallas as pl


def kernel(user_table, item_table, edge_index):
    raise NotImplementedError("write your pallas kernel here")


# speedup vs baseline: 1.2210x; 1.2210x over previous
import functools

import jax
import jax.numpy as jnp
from jax import lax
from jax.experimental import pallas as pl
from jax.experimental.pallas import tpu as pltpu
from jax.experimental.pallas import tpu_sc as plsc

NU, NI, D = 4000, 6000, 64
N = NU + NI
NPAD = 10240
E = 320000
NC, NS = 2, 16
NW = NC * NS
CHUNK = 128
EPT = 10240
EPAD = EPT * NW
NITER = EPT // CHUNK
RPT = NPAD // NS

_mesh = plsc.VectorSubcoreMesh(core_axis_name="c", subcore_axis_name="s")


@functools.partial(
    pl.kernel,
    out_type=jax.ShapeDtypeStruct((NC, NPAD, D), jnp.float32),
    mesh=_mesh,
    scratch_types=[
        pltpu.VMEM((CHUNK,), jnp.int32),
        pltpu.VMEM((CHUNK,), jnp.int32),
        pltpu.VMEM((CHUNK, D), jnp.float32),
        pltpu.VMEM_SHARED((NPAD, D), jnp.float32),
        pltpu.SemaphoreType.DMA,
    ],
    compiler_params=pltpu.CompilerParams(use_tc_tiling_on_sc=False),
)
def _sc_segsum(z_hbm, cols_hbm, rows_hbm, zeros_hbm, out_hbm,
               colv, rowv, gbuf, acc_sh, sem):
    cid = lax.axis_index("c")
    sid = lax.axis_index("s")
    wid = sid * NC + cid

    r0 = pl.multiple_of(sid * RPT, 8)
    pltpu.sync_copy(zeros_hbm.at[pl.ds(r0, RPT)], acc_sh.at[pl.ds(r0, RPT)])
    plsc.subcore_barrier()

    base = wid * EPT

    def step(i, c):
        st = pl.multiple_of(base + i * CHUNK, 8)
        pltpu.sync_copy(cols_hbm.at[pl.ds(st, CHUNK)], colv)
        pltpu.sync_copy(rows_hbm.at[pl.ds(st, CHUNK)], rowv)
        pltpu.async_copy(z_hbm.at[colv], gbuf, sem).wait()
        pltpu.sync_copy(gbuf, acc_sh.at[rowv], add=True)
        return c

    lax.fori_loop(0, NITER, step, 0)
    plsc.subcore_barrier()

    @pl.when(sid == 0)
    def _():
        pltpu.sync_copy(acc_sh, out_hbm.at[cid])


def _prep_body(degp_ref, u_ref, it_ref, z0_ref, s2b_ref):
    deg = degp_ref[0] + degp_ref[1] + 1.0
    s = lax.rsqrt(deg)
    s2b_ref[...] = 1.0 / deg
    z0_ref[pl.ds(0, NU)] = s[0:NU] * u_ref[...]
    z0_ref[pl.ds(NU, NI)] = s[NU:N] * it_ref[...]
    z0_ref[pl.ds(N, NPAD - N)] = jnp.zeros((NPAD - N, D), jnp.float32)


def _acc_body(p_ref, z_ref, s2b_ref, t_ref, zn_ref, tn_ref):
    zn = s2b_ref[...] * (p_ref[0] + p_ref[1] + z_ref[...])
    zn_ref[...] = zn
    tn_ref[...] = t_ref[...] + zn


def _final_body(p_ref, z_ref, s2b_ref, t_ref, u_ref, it_ref):
    zn = s2b_ref[...] * (p_ref[0] + p_ref[1] + z_ref[...])
    out = (t_ref[...] + zn) * lax.rsqrt(s2b_ref[...])
    u_ref[...] = out[0:NU]
    it_ref[...] = out[NU:N]


_prep = pl.pallas_call(
    _prep_body,
    out_shape=[jax.ShapeDtypeStruct((NPAD, D), jnp.float32),
               jax.ShapeDtypeStruct((NPAD, D), jnp.float32)],
)

_acc = pl.pallas_call(
    _acc_body,
    out_shape=[jax.ShapeDtypeStruct((NPAD, D), jnp.float32),
               jax.ShapeDtypeStruct((NPAD, D), jnp.float32)],
)

_final = pl.pallas_call(
    _final_body,
    out_shape=[jax.ShapeDtypeStruct((NU, D), jnp.float32),
               jax.ShapeDtypeStruct((NI, D), jnp.float32)],
)


def kernel(user_table, item_table, edge_index):
    keys = edge_index[0] * N + edge_index[1]
    sk = jnp.sort(keys)
    rows = sk // N
    cols = sk % N
    first = jnp.concatenate(
        [jnp.ones((1,), jnp.bool_), sk[1:] != sk[:-1]])
    rows_eff = jnp.where(first, rows, N)
    pad_r = jnp.full((EPAD - E,), N, jnp.int32)
    pad_c = jnp.zeros((EPAD - E,), jnp.int32)
    rows_p = jnp.concatenate([rows_eff, pad_r])
    cols_p = jnp.concatenate([cols, pad_c])

    zeros = jnp.zeros((NPAD, D), jnp.float32)
    ones_t = jnp.ones((NPAD, D), jnp.float32)

    degp = _sc_segsum(ones_t, cols_p, rows_p, zeros)
    z0, s2b = _prep(degp, user_table, item_table)
    p = _sc_segsum(z0, cols_p, rows_p, zeros)
    z1, t1 = _acc(p, z0, s2b, z0)
    p = _sc_segsum(z1, cols_p, rows_p, zeros)
    z2, t2 = _acc(p, z1, s2b, t1)
    p = _sc_segsum(z2, cols_p, rows_p, zeros)
    return _final(p, z2, s2b, t2)


# speedup vs baseline: 1.5874x; 1.3001x over previous
import functools

import jax
import jax.numpy as jnp
from jax import lax
from jax.experimental import pallas as pl
from jax.experimental.pallas import tpu as pltpu
from jax.experimental.pallas import tpu_sc as plsc

NU, NI, D = 4000, 6000, 64
N = NU + NI
NPAD = 10240
E = 320000
NC, NS = 2, 16
NW = NC * NS
CHUNK = 128
EPT = 10240
EPAD = EPT * NW
NITER = EPT // CHUNK
RPT = NPAD // NS

_mesh = plsc.VectorSubcoreMesh(core_axis_name="c", subcore_axis_name="s")


NBUF = 4
LEAD = 2


@functools.partial(
    pl.kernel,
    out_type=jax.ShapeDtypeStruct((NC, NPAD, D), jnp.float32),
    mesh=_mesh,
    scratch_types=[
        pltpu.VMEM((NITER, CHUNK), jnp.int32),
        pltpu.VMEM((NITER, CHUNK), jnp.int32),
        [pltpu.VMEM((CHUNK, D), jnp.float32)] * NBUF,
        [pltpu.SemaphoreType.DMA] * NBUF,
        [pltpu.SemaphoreType.DMA] * NBUF,
        pltpu.VMEM_SHARED((NPAD, D), jnp.float32),
        pltpu.SemaphoreType.DMA,
    ],
    compiler_params=pltpu.CompilerParams(use_tc_tiling_on_sc=False),
)
def _sc_segsum(z_hbm, cols_hbm, rows_hbm, zeros_hbm, out_hbm,
               colv, rowv, gbufs, gsems, ssems, acc_sh, sem):
    cid = lax.axis_index("c")
    sid = lax.axis_index("s")
    wid = sid * NC + cid

    r0 = pl.multiple_of(sid * RPT, 8)
    pltpu.sync_copy(zeros_hbm.at[pl.ds(r0, RPT)], acc_sh.at[pl.ds(r0, RPT)])
    pltpu.sync_copy(cols_hbm.at[wid], colv)
    pltpu.sync_copy(rows_hbm.at[wid], rowv)
    plsc.subcore_barrier()

    def gather(i, b):
        pltpu.async_copy(z_hbm.at[colv.at[i]], gbufs[b], gsems[b])

    def gather_wait(i, b):
        pltpu.make_async_copy(z_hbm.at[colv.at[i]], gbufs[b], gsems[b]).wait()

    def scat(i, b):
        pltpu.async_copy(gbufs[b], acc_sh.at[rowv.at[i]], ssems[b], add=True)

    def scat_wait(i, b):
        pltpu.make_async_copy(gbufs[b], acc_sh.at[rowv.at[i]],
                              ssems[b]).wait()

    for b in range(LEAD):
        gather(b, b)

    def step(j, c):
        for b in range(NBUF):
            i = j * NBUF + b
            gather_wait(i, b)
            scat(i, b)
            bn = (b + LEAD) % NBUF

            @pl.when(i - (NBUF - LEAD) >= 0)
            def _():
                scat_wait(i - (NBUF - LEAD), bn)

            @pl.when(i + LEAD < NITER)
            def _():
                gather(i + LEAD, bn)
        return c

    lax.fori_loop(0, NITER // NBUF, step, 0)
    for k in range(NITER - (NBUF - LEAD), NITER):
        scat_wait(k, k % NBUF)
    plsc.subcore_barrier()

    @pl.when(sid == 0)
    def _():
        pltpu.sync_copy(acc_sh, out_hbm.at[cid])


def _prep_body(degp_ref, u_ref, it_ref, z0_ref, s2b_ref):
    deg = degp_ref[0] + degp_ref[1] + 1.0
    s = lax.rsqrt(deg)
    s2b_ref[...] = 1.0 / deg
    z0_ref[pl.ds(0, NU)] = s[0:NU] * u_ref[...]
    z0_ref[pl.ds(NU, NI)] = s[NU:N] * it_ref[...]
    z0_ref[pl.ds(N, NPAD - N)] = jnp.zeros((NPAD - N, D), jnp.float32)


def _acc_body(p_ref, z_ref, s2b_ref, t_ref, zn_ref, tn_ref):
    zn = s2b_ref[...] * (p_ref[0] + p_ref[1] + z_ref[...])
    zn_ref[...] = zn
    tn_ref[...] = t_ref[...] + zn


def _final_body(p_ref, z_ref, s2b_ref, t_ref, u_ref, it_ref):
    zn = s2b_ref[...] * (p_ref[0] + p_ref[1] + z_ref[...])
    out = (t_ref[...] + zn) * lax.rsqrt(s2b_ref[...])
    u_ref[...] = out[0:NU]
    it_ref[...] = out[NU:N]


_prep = pl.pallas_call(
    _prep_body,
    out_shape=[jax.ShapeDtypeStruct((NPAD, D), jnp.float32),
               jax.ShapeDtypeStruct((NPAD, D), jnp.float32)],
)

_acc = pl.pallas_call(
    _acc_body,
    out_shape=[jax.ShapeDtypeStruct((NPAD, D), jnp.float32),
               jax.ShapeDtypeStruct((NPAD, D), jnp.float32)],
)

_final = pl.pallas_call(
    _final_body,
    out_shape=[jax.ShapeDtypeStruct((NU, D), jnp.float32),
               jax.ShapeDtypeStruct((NI, D), jnp.float32)],
)


def kernel(user_table, item_table, edge_index):
    keys = edge_index[0] * N + edge_index[1]
    sk = jnp.sort(keys)
    rows = sk // N
    cols = sk % N
    first = jnp.concatenate(
        [jnp.ones((1,), jnp.bool_), sk[1:] != sk[:-1]])
    rows_eff = jnp.where(first, rows, N)
    pad_r = jnp.full((EPAD - E,), N, jnp.int32)
    pad_c = jnp.zeros((EPAD - E,), jnp.int32)
    rows_p = jnp.concatenate([rows_eff, pad_r]).reshape(NW, NITER, CHUNK)
    cols_p = jnp.concatenate([cols, pad_c]).reshape(NW, NITER, CHUNK)

    zeros = jnp.zeros((NPAD, D), jnp.float32)
    ones_t = jnp.ones((NPAD, D), jnp.float32)

    degp = _sc_segsum(ones_t, cols_p, rows_p, zeros)
    z0, s2b = _prep(degp, user_table, item_table)
    p = _sc_segsum(z0, cols_p, rows_p, zeros)
    z1, t1 = _acc(p, z0, s2b, z0)
    p = _sc_segsum(z1, cols_p, rows_p, zeros)
    z2, t2 = _acc(p, z1, s2b, t1)
    p = _sc_segsum(z2, cols_p, rows_p, zeros)
    return _final(p, z2, s2b, t2)


# speedup vs baseline: 1.6798x; 1.0582x over previous
import functools

import jax
import jax.numpy as jnp
from jax import lax
from jax.experimental import pallas as pl
from jax.experimental.pallas import tpu as pltpu
from jax.experimental.pallas import tpu_sc as plsc

NU, NI, D = 4000, 6000, 64
N = NU + NI
NPAD = 10240
E = 320000
NC, NS = 2, 16
NW = NC * NS
CHUNK = 128
EPT = 10240
EPAD = EPT * NW
NITER = EPT // CHUNK
RPT = NPAD // NS

_mesh = plsc.VectorSubcoreMesh(core_axis_name="c", subcore_axis_name="s")


NBUF = 8
LEAD = 4


@functools.partial(
    pl.kernel,
    out_type=jax.ShapeDtypeStruct((NC, NPAD, D), jnp.float32),
    mesh=_mesh,
    scratch_types=[
        pltpu.VMEM((NITER, CHUNK), jnp.int32),
        pltpu.VMEM((NITER, CHUNK), jnp.int32),
        [pltpu.VMEM((CHUNK, D), jnp.float32)] * NBUF,
        [pltpu.SemaphoreType.DMA] * NBUF,
        [pltpu.SemaphoreType.DMA] * NBUF,
        pltpu.VMEM_SHARED((NPAD, D), jnp.float32),
        pltpu.SemaphoreType.DMA,
    ],
    compiler_params=pltpu.CompilerParams(use_tc_tiling_on_sc=False),
)
def _sc_segsum(z_hbm, cols_hbm, rows_hbm, out_hbm,
               colv, rowv, gbufs, gsems, ssems, acc_sh, sem):
    cid = lax.axis_index("c")
    sid = lax.axis_index("s")
    wid = sid * NC + cid

    r0 = pl.multiple_of(sid * RPT, 8)
    pltpu.sync_copy(z_hbm.at[pl.ds(r0, RPT)], acc_sh.at[pl.ds(r0, RPT)])
    pltpu.sync_copy(cols_hbm.at[wid], colv)
    pltpu.sync_copy(rows_hbm.at[wid], rowv)
    plsc.subcore_barrier()

    def gather(i, b):
        pltpu.async_copy(z_hbm.at[colv.at[i]], gbufs[b], gsems[b])

    def gather_wait(i, b):
        pltpu.make_async_copy(z_hbm.at[colv.at[i]], gbufs[b], gsems[b]).wait()

    def scat(i, b):
        pltpu.async_copy(gbufs[b], acc_sh.at[rowv.at[i]], ssems[b], add=True)

    def scat_wait(i, b):
        pltpu.make_async_copy(gbufs[b], acc_sh.at[rowv.at[i]],
                              ssems[b]).wait()

    for b in range(LEAD):
        gather(b, b)

    def step(j, c):
        for b in range(NBUF):
            i = j * NBUF + b
            gather_wait(i, b)
            scat(i, b)
            bn = (b + LEAD) % NBUF

            @pl.when(i - (NBUF - LEAD) >= 0)
            def _():
                scat_wait(i - (NBUF - LEAD), bn)

            @pl.when(i + LEAD < NITER)
            def _():
                gather(i + LEAD, bn)
        return c

    lax.fori_loop(0, NITER // NBUF, step, 0)
    for k in range(NITER - (NBUF - LEAD), NITER):
        scat_wait(k, k % NBUF)
    plsc.subcore_barrier()

    @pl.when(sid == 0)
    def _():
        pltpu.sync_copy(acc_sh, out_hbm.at[cid])


def _prep_body(degp_ref, u_ref, it_ref, z0_ref, s2b_ref):
    deg = degp_ref[0] + degp_ref[1] - 1.0
    s = lax.rsqrt(deg)
    s2b_ref[...] = 1.0 / deg
    z0_ref[pl.ds(0, NU)] = s[0:NU] * u_ref[...]
    z0_ref[pl.ds(NU, NI)] = s[NU:N] * it_ref[...]
    z0_ref[pl.ds(N, NPAD - N)] = jnp.zeros((NPAD - N, D), jnp.float32)


def _acc_body(p_ref, z_ref, s2b_ref, t_ref, zn_ref, tn_ref):
    zn = s2b_ref[...] * (p_ref[0] + p_ref[1] - z_ref[...])
    zn_ref[...] = zn
    tn_ref[...] = t_ref[...] + zn


def _final_body(p_ref, z_ref, s2b_ref, t_ref, u_ref, it_ref):
    zn = s2b_ref[...] * (p_ref[0] + p_ref[1] - z_ref[...])
    out = (t_ref[...] + zn) * lax.rsqrt(s2b_ref[...])
    u_ref[...] = out[0:NU]
    it_ref[...] = out[NU:N]


_prep = pl.pallas_call(
    _prep_body,
    out_shape=[jax.ShapeDtypeStruct((NPAD, D), jnp.float32),
               jax.ShapeDtypeStruct((NPAD, D), jnp.float32)],
)

_acc = pl.pallas_call(
    _acc_body,
    out_shape=[jax.ShapeDtypeStruct((NPAD, D), jnp.float32),
               jax.ShapeDtypeStruct((NPAD, D), jnp.float32)],
)

_final = pl.pallas_call(
    _final_body,
    out_shape=[jax.ShapeDtypeStruct((NU, D), jnp.float32),
               jax.ShapeDtypeStruct((NI, D), jnp.float32)],
)


def kernel(user_table, item_table, edge_index):
    keys = edge_index[0] * N + edge_index[1]
    sk = jnp.sort(keys)
    rows = sk // N
    cols = sk % N
    first = jnp.concatenate(
        [jnp.ones((1,), jnp.bool_), sk[1:] != sk[:-1]])
    rows_eff = jnp.where(first, rows, N)
    pad_r = jnp.full((EPAD - E,), N, jnp.int32)
    pad_c = jnp.zeros((EPAD - E,), jnp.int32)
    rows_p = jnp.concatenate([rows_eff, pad_r]).reshape(NW, NITER, CHUNK)
    cols_p = jnp.concatenate([cols, pad_c]).reshape(NW, NITER, CHUNK)

    ones_t = jnp.ones((NPAD, D), jnp.float32)

    degp = _sc_segsum(ones_t, cols_p, rows_p)
    z0, s2b = _prep(degp, user_table, item_table)
    p = _sc_segsum(z0, cols_p, rows_p)
    z1, t1 = _acc(p, z0, s2b, z0)
    p = _sc_segsum(z1, cols_p, rows_p)
    z2, t2 = _acc(p, z1, s2b, t1)
    p = _sc_segsum(z2, cols_p, rows_p)
    return _final(p, z2, s2b, t2)


# speedup vs baseline: 3.0600x; 1.8216x over previous
import functools

import jax
import jax.numpy as jnp
from jax import lax
from jax.experimental import pallas as pl
from jax.experimental.pallas import tpu as pltpu
from jax.experimental.pallas import tpu_sc as plsc

NU, NI, D = 4000, 6000, 64
N = NU + NI
NPAD = 10240
E = 320000
NC, NS = 2, 16
NH = 5120
NHALF = 5000
LTRASH = 5000
CHUNK = 128
EPT = 10752
NITER = EPT // CHUNK
CAP = NS * EPT
RPT = NPAD // NS
APT = NH // NS

NBUF = 4
LEAD = 2

_mesh = plsc.VectorSubcoreMesh(core_axis_name="c", subcore_axis_name="s")


@functools.partial(
    pl.kernel,
    out_type=jax.ShapeDtypeStruct((NC, NH, D), jnp.float32),
    mesh=_mesh,
    scratch_types=[
        pltpu.VMEM((NITER, CHUNK), jnp.int32),
        pltpu.VMEM((NITER, CHUNK), jnp.int32),
        [pltpu.VMEM((CHUNK, D), jnp.float32)] * NBUF,
        [pltpu.SemaphoreType.DMA] * NBUF,
        [pltpu.SemaphoreType.DMA] * NBUF,
        pltpu.VMEM_SHARED((NH, D), jnp.float32),
        pltpu.VMEM_SHARED((NPAD, D), jnp.float32),
    ],
    compiler_params=pltpu.CompilerParams(use_tc_tiling_on_sc=False),
)
def _sc_segsum(z_hbm, cols_hbm, rows_hbm, out_hbm,
               colv, rowv, gbufs, gsems, ssems, acc_sh, z_sh):
    cid = lax.axis_index("c")
    sid = lax.axis_index("s")
    blk = cid * NS + sid

    zr = pl.multiple_of(sid * RPT, 8)
    pltpu.sync_copy(z_hbm.at[pl.ds(zr, RPT)], z_sh.at[pl.ds(zr, RPT)])
    ar = pl.multiple_of(sid * APT, 8)
    zsrc = pl.multiple_of(cid * NHALF + sid * APT, 8)
    pltpu.sync_copy(z_hbm.at[pl.ds(zsrc, APT)], acc_sh.at[pl.ds(ar, APT)])
    pltpu.sync_copy(cols_hbm.at[blk], colv)
    pltpu.sync_copy(rows_hbm.at[blk], rowv)
    plsc.subcore_barrier()

    def gather(i, b):
        pltpu.async_copy(z_sh.at[colv.at[i]], gbufs[b], gsems[b])

    def gather_wait(i, b):
        pltpu.make_async_copy(z_sh.at[colv.at[i]], gbufs[b], gsems[b]).wait()

    def scat(i, b):
        pltpu.async_copy(gbufs[b], acc_sh.at[rowv.at[i]], ssems[b], add=True)

    def scat_wait(i, b):
        pltpu.make_async_copy(gbufs[b], acc_sh.at[rowv.at[i]],
                              ssems[b]).wait()

    for b in range(LEAD):
        gather(b, b)

    def step(j, c):
        for b in range(NBUF):
            i = j * NBUF + b
            gather_wait(i, b)
            scat(i, b)
            bn = (b + LEAD) % NBUF

            @pl.when(i - (NBUF - LEAD) >= 0)
            def _():
                scat_wait(i - (NBUF - LEAD), bn)

            @pl.when(i + LEAD < NITER)
            def _():
                gather(i + LEAD, bn)
        return c

    lax.fori_loop(0, NITER // NBUF, step, 0)
    for k in range(NITER - (NBUF - LEAD), NITER):
        scat_wait(k, k % NBUF)
    plsc.subcore_barrier()

    @pl.when(sid == 0)
    def _():
        pltpu.sync_copy(acc_sh, out_hbm.at[cid])


def _prep_body(degp_ref, u_ref, it_ref, z0_ref, s2b_ref):
    deg = jnp.concatenate([degp_ref[0, 0:NHALF], degp_ref[1, 0:NHALF]],
                          axis=0)
    s = lax.rsqrt(deg)
    s2b_ref[pl.ds(0, N)] = 1.0 / deg
    s2b_ref[pl.ds(N, NPAD - N)] = jnp.ones((NPAD - N, D), jnp.float32)
    z0_ref[pl.ds(0, NU)] = s[0:NU] * u_ref[...]
    z0_ref[pl.ds(NU, NI)] = s[NU:N] * it_ref[...]
    z0_ref[pl.ds(N, NPAD - N)] = jnp.zeros((NPAD - N, D), jnp.float32)


def _acc_body(p_ref, s2b_ref, t_ref, zn_ref, tn_ref):
    az = jnp.concatenate([p_ref[0, 0:NHALF], p_ref[1, 0:NHALF]], axis=0)
    zn = s2b_ref[0:N] * az
    zn_ref[pl.ds(0, N)] = zn
    zn_ref[pl.ds(N, NPAD - N)] = jnp.zeros((NPAD - N, D), jnp.float32)
    tn_ref[pl.ds(0, N)] = t_ref[0:N] + zn
    tn_ref[pl.ds(N, NPAD - N)] = jnp.zeros((NPAD - N, D), jnp.float32)


def _final_body(p_ref, s2b_ref, t_ref, u_ref, it_ref):
    az = jnp.concatenate([p_ref[0, 0:NHALF], p_ref[1, 0:NHALF]], axis=0)
    s2 = s2b_ref[0:N]
    out = (t_ref[0:N] + s2 * az) * lax.rsqrt(s2)
    u_ref[...] = out[0:NU]
    it_ref[...] = out[NU:N]


_prep = pl.pallas_call(
    _prep_body,
    out_shape=[jax.ShapeDtypeStruct((NPAD, D), jnp.float32),
               jax.ShapeDtypeStruct((NPAD, D), jnp.float32)],
)

_acc = pl.pallas_call(
    _acc_body,
    out_shape=[jax.ShapeDtypeStruct((NPAD, D), jnp.float32),
               jax.ShapeDtypeStruct((NPAD, D), jnp.float32)],
)

_final = pl.pallas_call(
    _final_body,
    out_shape=[jax.ShapeDtypeStruct((NU, D), jnp.float32),
               jax.ShapeDtypeStruct((NI, D), jnp.float32)],
)


def kernel(user_table, item_table, edge_index):
    keys = edge_index[0] * N + edge_index[1]
    sk = jnp.sort(keys)
    rows = sk // N
    cols = sk % N
    first = jnp.concatenate(
        [jnp.ones((1,), jnp.bool_), sk[1:] != sk[:-1]])
    rows_eff = jnp.where(first, rows, N)

    c0 = jnp.searchsorted(rows, NHALF).astype(jnp.int32)
    r_blk0 = rows_eff[0:CAP]
    c_blk0 = cols[0:CAP]
    lrow0 = jnp.where(r_blk0 < NHALF, r_blk0, LTRASH)
    r_pad = jnp.concatenate([rows_eff, jnp.full((CAP,), N, jnp.int32)])
    c_pad = jnp.concatenate([cols, jnp.zeros((CAP,), jnp.int32)])
    r_blk1 = lax.dynamic_slice(r_pad, (c0,), (CAP,))
    c_blk1 = lax.dynamic_slice(c_pad, (c0,), (CAP,))
    lrow1 = jnp.where((r_blk1 >= NHALF) & (r_blk1 < N),
                      r_blk1 - NHALF, LTRASH)
    rows_p = jnp.stack([lrow0, lrow1]).reshape(NC * NS, NITER, CHUNK)
    cols_p = jnp.stack([c_blk0, c_blk1]).reshape(NC * NS, NITER, CHUNK)

    ones_t = jnp.ones((NPAD, D), jnp.float32)

    degp = _sc_segsum(ones_t, cols_p, rows_p)
    z0, s2b = _prep(degp, user_table, item_table)
    p = _sc_segsum(z0, cols_p, rows_p)
    z1, t1 = _acc(p, s2b, z0)
    p = _sc_segsum(z1, cols_p, rows_p)
    z2, t2 = _acc(p, s2b, t1)
    p = _sc_segsum(z2, cols_p, rows_p)
    return _final(p, s2b, t2)
